# baseline (speedup 1.0000x reference)
import functools

import jax
import jax.numpy as jnp
from jax import lax
from jax.experimental import pallas as pl
from jax.experimental.pallas import tpu as pltpu
from jax.experimental.pallas import tpu_sc as plsc

N_CODES = 8192
DIM = 32
ROWS = 4096
HALF = ROWS // 2
ROW_BLK = 512
COL_BLK = 2048
N_ROW_BLKS = HALF // ROW_BLK
N_COL_BLKS = N_CODES // COL_BLK
BETA = 0.25


LANES = 128
RSUB = 64


def _dist_argmin_body(z_ref, cbt_ref, cn_ref, idx_ref, loss_ref, acc_ref):
    i = pl.program_id(0)

    @pl.when(i == 0)
    def _init():
        acc_ref[...] = jnp.zeros((1, 1), dtype=jnp.float32)

    zb = z_ref[...]
    znorm = jnp.sum(zb * zb, axis=1, keepdims=True)
    prods = [
        lax.dot_general(
            zb + zb, cbt_ref[:, c * COL_BLK:(c + 1) * COL_BLK],
            dimension_numbers=(((1,), (0,)), ((), ())),
            preferred_element_type=jnp.float32,
        )
        for c in range(N_COL_BLKS)
    ]

    gpc = COL_BLK // LANES
    step_sum = jnp.float32(0.0)
    for rs in range(ROW_BLK // RSUB):
        zn = znorm[rs * RSUB:(rs + 1) * RSUB, :]
        lane_min = jnp.full((RSUB, LANES), jnp.inf, dtype=jnp.float32)
        lane_grp = jnp.zeros((RSUB, LANES), dtype=jnp.int32)
        for c in range(N_COL_BLKS):
            pc = prods[c]
            for g in range(gpc):
                gg = c * gpc + g
                d = (zn + cn_ref[:, pl.ds(gg * LANES, LANES)]) \
                    - pc[rs * RSUB:(rs + 1) * RSUB, g * LANES:(g + 1) * LANES]
                better = d < lane_min
                lane_min = jnp.where(better, d, lane_min)
                lane_grp = jnp.where(better, gg, lane_grp)
        row_min = jnp.min(lane_min, axis=1, keepdims=True)
        lane_iota = lax.broadcasted_iota(jnp.int32, (RSUB, LANES), 1)
        cand = jnp.where(lane_min == row_min, lane_grp * LANES + lane_iota,
                         jnp.int32(N_CODES))
        idx_ref[pl.ds(rs * RSUB, RSUB)] = jnp.min(cand, axis=1)
        step_sum = step_sum + jnp.sum(row_min)

    acc = acc_ref[0, 0] + step_sum
    acc_ref[...] = acc.reshape(1, 1)
    loss_ref[...] = acc.reshape(1, 1)


def _dist_argmin(z_flat, codebook_t, cn):
    return pl.pallas_call(
        _dist_argmin_body,
        grid=(N_ROW_BLKS,),
        in_specs=[
            pl.BlockSpec((ROW_BLK, DIM), lambda i: (i, 0)),
            pl.BlockSpec((DIM, N_CODES), lambda i: (0, 0)),
            pl.BlockSpec((1, N_CODES), lambda i: (0, 0)),
        ],
        out_specs=[
            pl.BlockSpec((ROW_BLK,), lambda i: (i,)),
            pl.BlockSpec((1, 1), lambda i: (0, 0)),
        ],
        out_shape=[
            jax.ShapeDtypeStruct((HALF,), jnp.int32),
            jax.ShapeDtypeStruct((1, 1), jnp.float32),
        ],
        scratch_shapes=[
            pltpu.VMEM((1, 1), jnp.float32),
        ],
    )(z_flat, codebook_t, cn)


def _sc_gather_half(codebook_grp, idx_h, b, ch, h, w):
    info = plsc.get_sparse_core_info()
    nc, ns = info.num_cores, info.num_subcores
    nw = nc * ns
    b_per_w = HALF // nw
    n_rc = b_per_w // 16
    h_per_w = b_per_w // w
    merge = 8 // h_per_w
    mesh = plsc.VectorSubcoreMesh(core_axis_name="c", subcore_axis_name="s")

    @functools.partial(
        pl.kernel, mesh=mesh,
        out_type=[
            jax.ShapeDtypeStruct((b // 2, ch, h, w), jnp.float32),
            jax.ShapeDtypeStruct((b // 2, 1, h, w), jnp.int32),
        ],
        compiler_params=pltpu.CompilerParams(needs_layout_passes=False),
        scratch_types=[
            pltpu.VMEM((b_per_w,), jnp.int32),
            pltpu.VMEM((b_per_w,), jnp.int32),
            pltpu.VMEM((b_per_w, 128), jnp.float32),
            pltpu.VMEM((DIM, 8, w), jnp.float32),
            pltpu.VMEM((8, w), jnp.int32),
            pltpu.VMEM_SHARED((ns, DIM, h_per_w, w), jnp.float32),
            pltpu.VMEM_SHARED((ns, h_per_w, w), jnp.int32),
            pltpu.SemaphoreType.DMA,
        ],
    )
    def k(table_hbm, idx_hbm, zq_hbm, ind_hbm,
          idx_v, idxg_v, grp_v, out8_v, idx8_v, zq_s, idx_s, sem):
        cid = lax.axis_index("c")
        sid = lax.axis_index("s")
        wid = cid * ns + sid
        base = wid * b_per_w
        part = (sid % merge) * h_per_w
        pltpu.sync_copy(idx_hbm.at[pl.ds(base, b_per_w)], idx_v)
        for rc in range(n_rc):
            sl = pl.ds(rc * 16, 16)
            idxg_v[sl] = lax.shift_right_logical(idx_v[sl], 2)
        pltpu.async_copy(table_hbm.at[idxg_v], grp_v, sem).wait()
        for rc in range(n_rc):
            sl = pl.ds(rc * 16, 16)
            rows16 = lax.iota(jnp.int32, 16) + (rc * 16)
            hh16 = lax.shift_right_logical(rows16, 5) + part
            ww16 = rows16 & (w - 1)
            i16 = idx_v[sl]
            plsc.store_scatter(idx8_v, [hh16, ww16], i16)
            off16 = (i16 & 3) * DIM
            for j in range(DIM):
                vals = plsc.load_gather(grp_v, [rows16, off16 + j])
                plsc.store_scatter(
                    out8_v, [jnp.full((16,), j, jnp.int32), hh16, ww16], vals)

        @pl.when(sid % merge != 0)
        def _stage():
            pltpu.sync_copy(out8_v.at[:, pl.ds(part, h_per_w), :],
                            zq_s.at[sid])
            pltpu.sync_copy(idx8_v.at[pl.ds(part, h_per_w), :],
                            idx_s.at[sid])

        plsc.subcore_barrier()

        @pl.when(sid % merge == 0)
        def _store():
            bb = base // (h * w)
            h0 = pl.multiple_of((base % (h * w)) // w, 8)
            for u in range(1, merge):
                pltpu.sync_copy(
                    zq_s.at[sid + u],
                    out8_v.at[:, pl.ds(u * h_per_w, h_per_w), :])
                pltpu.sync_copy(
                    idx_s.at[sid + u],
                    idx8_v.at[pl.ds(u * h_per_w, h_per_w), :])
            pltpu.sync_copy(out8_v, zq_hbm.at[bb, :, pl.ds(h0, 8), :])
            pltpu.sync_copy(idx8_v, ind_hbm.at[bb, 0, pl.ds(h0, 8), :])

    return k(codebook_grp, idx_h)


def kernel(z, codebook):
    b, ch, h, w = z.shape
    zp = jnp.transpose(z, (0, 2, 3, 1))
    z_flat = zp.reshape(-1, DIM)

    cn = jnp.sum(codebook ** 2, axis=1).reshape(1, N_CODES)
    cbt = codebook.T
    table = codebook.reshape(N_CODES // 4, 128)

    idx_a, acc_a = _dist_argmin(z_flat[:HALF], cbt, cn)
    zq_a, ind_a = _sc_gather_half(table, idx_a, b, ch, h, w)
    idx_b, acc_b = _dist_argmin(z_flat[HALF:], cbt, cn)
    zq_b, ind_b = _sc_gather_half(table, idx_b, b, ch, h, w)

    m = (acc_a[0, 0] + acc_b[0, 0]) / jnp.float32(ROWS * DIM)
    codebook_loss = m + m * BETA
    zq = jnp.concatenate([zq_a, zq_b], axis=0)
    ind = jnp.concatenate([ind_a, ind_b], axis=0)
    return zq, codebook_loss, ind


# speedup vs baseline: 1.1204x; 1.1204x over previous
import functools

import jax
import jax.numpy as jnp
from jax import lax
from jax.experimental import pallas as pl
from jax.experimental.pallas import tpu as pltpu
from jax.experimental.pallas import tpu_sc as plsc

N_CODES = 8192
DIM = 32
ROWS = 4096
ROW_BLK = 512
COL_BLK = 2048
N_ROW_BLKS = ROWS // ROW_BLK
N_COL_BLKS = N_CODES // COL_BLK
BETA = 0.25


LANES = 128
RSUB = 64


def _dist_argmin_body(z_ref, cbt_ref, cn_ref, idx_ref, loss_ref, acc_ref):
    i = pl.program_id(0)

    @pl.when(i == 0)
    def _init():
        acc_ref[...] = jnp.zeros((1, 1), dtype=jnp.float32)

    zb = z_ref[...]
    znorm = jnp.sum(zb * zb, axis=1, keepdims=True)
    prods = [
        lax.dot_general(
            zb + zb, cbt_ref[:, c * COL_BLK:(c + 1) * COL_BLK],
            dimension_numbers=(((1,), (0,)), ((), ())),
            preferred_element_type=jnp.float32,
        )
        for c in range(N_COL_BLKS)
    ]

    gpc = COL_BLK // LANES
    step_sum = jnp.float32(0.0)
    for rs in range(ROW_BLK // RSUB):
        zn = znorm[rs * RSUB:(rs + 1) * RSUB, :]
        lane_min = jnp.full((RSUB, LANES), jnp.inf, dtype=jnp.float32)
        lane_grp = jnp.zeros((RSUB, LANES), dtype=jnp.int32)
        for c in range(N_COL_BLKS):
            pc = prods[c]
            for g in range(gpc):
                gg = c * gpc + g
                d = (zn + cn_ref[:, pl.ds(gg * LANES, LANES)]) \
                    - pc[rs * RSUB:(rs + 1) * RSUB, g * LANES:(g + 1) * LANES]
                better = d < lane_min
                lane_min = jnp.where(better, d, lane_min)
                lane_grp = jnp.where(better, gg, lane_grp)
        row_min = jnp.min(lane_min, axis=1, keepdims=True)
        lane_iota = lax.broadcasted_iota(jnp.int32, (RSUB, LANES), 1)
        cand = jnp.where(lane_min == row_min, lane_grp * LANES + lane_iota,
                         jnp.int32(N_CODES))
        idx_ref[pl.ds(rs * RSUB, RSUB)] = jnp.min(cand, axis=1)
        step_sum = step_sum + jnp.sum(row_min)

    acc = acc_ref[0, 0] + step_sum
    acc_ref[...] = acc.reshape(1, 1)

    @pl.when(i == N_ROW_BLKS - 1)
    def _loss():
        m = acc / jnp.float32(ROWS * DIM)
        loss_ref[...] = (m + m * BETA).reshape(1, 1)


def _dist_argmin(z_flat, codebook_t, cn):
    return pl.pallas_call(
        _dist_argmin_body,
        grid=(N_ROW_BLKS,),
        in_specs=[
            pl.BlockSpec((ROW_BLK, DIM), lambda i: (i, 0)),
            pl.BlockSpec((DIM, N_CODES), lambda i: (0, 0)),
            pl.BlockSpec((1, N_CODES), lambda i: (0, 0)),
        ],
        out_specs=[
            pl.BlockSpec((ROW_BLK,), lambda i: (i,)),
            pl.BlockSpec((1, 1), lambda i: (0, 0)),
        ],
        out_shape=[
            jax.ShapeDtypeStruct((ROWS,), jnp.int32),
            jax.ShapeDtypeStruct((1, 1), jnp.float32),
        ],
        scratch_shapes=[
            pltpu.VMEM((1, 1), jnp.float32),
        ],
    )(z_flat, codebook_t, cn)


def _sc_gather(codebook_grp, idx, b, ch, h, w):
    info = plsc.get_sparse_core_info()
    nc, ns = info.num_cores, info.num_subcores
    nw = nc * ns
    b_per_w = ROWS // nw
    n_rc = b_per_w // 16
    h_per_w = b_per_w // w
    mesh = plsc.VectorSubcoreMesh(core_axis_name="c", subcore_axis_name="s")

    @functools.partial(
        pl.kernel, mesh=mesh,
        out_type=[
            jax.ShapeDtypeStruct((b, ch, h, w), jnp.float32),
            jax.ShapeDtypeStruct((b, 1, h, w), jnp.int32),
        ],
        compiler_params=pltpu.CompilerParams(needs_layout_passes=False),
        scratch_types=[
            pltpu.VMEM((b_per_w,), jnp.int32),
            pltpu.VMEM((b_per_w,), jnp.int32),
            pltpu.VMEM((b_per_w, 128), jnp.float32),
            pltpu.VMEM((DIM, 2 * h_per_w, w), jnp.float32),
            pltpu.VMEM((2 * h_per_w, w), jnp.int32),
            pltpu.VMEM_SHARED((ns, DIM, h_per_w, w), jnp.float32),
            pltpu.VMEM_SHARED((ns, h_per_w, w), jnp.int32),
            pltpu.SemaphoreType.DMA,
        ],
    )
    def k(table_hbm, idx_hbm, zq_hbm, ind_hbm,
          idx_v, idxg_v, grp_v, out8_v, idx8_v, zq_s, idx_s, sem):
        cid = lax.axis_index("c")
        sid = lax.axis_index("s")
        wid = cid * ns + sid
        base = wid * b_per_w
        half = (sid % 2) * h_per_w
        pltpu.sync_copy(idx_hbm.at[pl.ds(base, b_per_w)], idx_v)
        for rc in range(n_rc):
            sl = pl.ds(rc * 16, 16)
            idxg_v[sl] = lax.shift_right_logical(idx_v[sl], 2)
        pltpu.async_copy(table_hbm.at[idxg_v], grp_v, sem).wait()
        for rc in range(n_rc):
            sl = pl.ds(rc * 16, 16)
            rows16 = lax.iota(jnp.int32, 16) + (rc * 16)
            hh16 = lax.shift_right_logical(rows16, 5) + half
            ww16 = rows16 & (w - 1)
            i16 = idx_v[sl]
            plsc.store_scatter(idx8_v, [hh16, ww16], i16)
            off16 = (i16 & 3) * DIM
            for j in range(DIM):
                vals = plsc.load_gather(grp_v, [rows16, off16 + j])
                plsc.store_scatter(
                    out8_v, [jnp.full((16,), j, jnp.int32), hh16, ww16], vals)

        @pl.when(sid % 2 == 1)
        def _stage():
            pltpu.sync_copy(out8_v.at[:, pl.ds(h_per_w, h_per_w), :],
                            zq_s.at[sid])
            pltpu.sync_copy(idx8_v.at[pl.ds(h_per_w, h_per_w), :],
                            idx_s.at[sid])

        plsc.subcore_barrier()

        @pl.when(sid % 2 == 0)
        def _store():
            bb = base // (h * w)
            h0 = pl.multiple_of((base % (h * w)) // w, 2 * h_per_w)
            pltpu.sync_copy(zq_s.at[sid + 1],
                            out8_v.at[:, pl.ds(h_per_w, h_per_w), :])
            pltpu.sync_copy(idx_s.at[sid + 1],
                            idx8_v.at[pl.ds(h_per_w, h_per_w), :])
            pltpu.sync_copy(out8_v, zq_hbm.at[bb, :, pl.ds(h0, 2 * h_per_w), :])
            pltpu.sync_copy(idx8_v, ind_hbm.at[bb, 0, pl.ds(h0, 2 * h_per_w), :])

    return k(codebook_grp, idx)


def kernel(z, codebook):
    b, ch, h, w = z.shape
    zp = jnp.transpose(z, (0, 2, 3, 1))
    z_flat = zp.reshape(-1, DIM)

    cn = jnp.sum(codebook ** 2, axis=1).reshape(1, N_CODES)
    idx, loss_sum = _dist_argmin(z_flat, codebook.T, cn)

    z_q_out, indices_out = _sc_gather(
        codebook.reshape(N_CODES // 4, 128), idx, b, ch, h, w)

    codebook_loss = loss_sum[0, 0]
    return z_q_out, codebook_loss, indices_out
